# baseline (speedup 1.0000x reference)
import jax
import jax.numpy as jnp
from jax import lax
from jax.experimental import pallas as pl
from jax.experimental.pallas import tpu as pltpu
from jax.experimental.pallas import tpu_sc as plsc

N = 100000
WIDTH = 128
MAX_DEGREE = 4
L = 16
NW = 32
RPT = N // NW
CHUNK = 125
CPAD = 128
NCHUNK = RPT // CHUNK
DBUF = 3152
DIO = 3136
GROUPS = WIDTH // L
RGRP = CPAD // L
NBUF = 3


def _sc_body(x_hbm, d_hbm, scale_hbm, out_hbm,
             scv, dv, xb0, xb1, xb2, ob0, ob1, ob2,
             in_sem0, in_sem1, in_sem2, out_sem0, out_sem1, out_sem2,
             d_sem, sc_sem):
    cid = lax.axis_index("c")
    sid = lax.axis_index("s")
    wid = sid * 2 + cid
    base = wid * RPT
    ab = jnp.minimum((base // 8) * 8, N - DIO)
    off = base - ab

    xbs = [xb0, xb1, xb2]
    obs = [ob0, ob1, ob2]
    in_sems = [in_sem0, in_sem1, in_sem2]
    out_sems = [out_sem0, out_sem1, out_sem2]

    def in_copy(b, ch):
        return pltpu.make_async_copy(
            x_hbm.at[pl.ds(base + ch * CHUNK, CHUNK)],
            xbs[b].at[pl.ds(0, CHUNK)], in_sems[b])

    def out_copy(b, ch):
        return pltpu.make_async_copy(
            obs[b].at[pl.ds(0, CHUNK)],
            out_hbm.at[pl.ds(base + ch * CHUNK, CHUNK)], out_sems[b])

    for b in range(NBUF):
        in_copy(b, b).start()

    d_dma = pltpu.make_async_copy(d_hbm.at[pl.ds(ab, DIO)],
                                  dv.at[pl.ds(0, DIO)], d_sem)
    sc_dma = pltpu.make_async_copy(scale_hbm, scv, sc_sem)
    d_dma.start()
    sc_dma.start()
    sc_dma.wait()
    esc = [[jnp.exp(scv[i, pl.ds(j * L, L)]) for j in range(GROUPS)]
           for i in range(MAX_DEGREE)]
    d_dma.wait()

    def compute(xbuf, obuf, ch):
        dbase = off + ch * CHUNK

        def grp(g, carry):
            drv = dv[pl.ds(dbase + g * L, L)]
            for k in range(L):
                dr = drv[k]
                b0 = dr == 0
                b1 = dr == 1
                b2 = dr == 2
                r = g * L + k
                for j in range(GROUPS):
                    m = jnp.where(b0, esc[0][j],
                                  jnp.where(b1, esc[1][j],
                                            jnp.where(b2, esc[2][j],
                                                      esc[3][j])))
                    obuf[r, pl.ds(j * L, L)] = xbuf[r, pl.ds(j * L, L)] * m
            return carry

        lax.fori_loop(0, RGRP, grp, 0)

    def round_body(i, carry):
        for b in range(NBUF):
            ch = NBUF * i + b
            in_copy(b, ch).wait()

            @pl.when(i > 0)
            def _():
                out_copy(b, ch - NBUF).wait()

            compute(xbs[b], obs[b], ch)
            out_copy(b, ch).start()

            @pl.when(ch + NBUF < NCHUNK)
            def _():
                in_copy(b, ch + NBUF).start()
        return carry

    lax.fori_loop(0, NCHUNK // NBUF, round_body, 0)

    last = NCHUNK - 1
    in_copy(0, last).wait()
    out_copy(0, last - NBUF).wait()
    compute(xbs[0], obs[0], last)
    out_copy(0, last).start()
    out_copy(0, last).wait()
    for b in range(1, NBUF):
        out_copy(b, last - NBUF + b).wait()


def kernel(x, d, scale):
    d32 = d.astype(jnp.int32)
    mesh = plsc.VectorSubcoreMesh(core_axis_name="c", subcore_axis_name="s")
    f = pl.kernel(
        _sc_body,
        out_type=jax.ShapeDtypeStruct((N, WIDTH), jnp.float32),
        mesh=mesh,
        scratch_types=[
            pltpu.VMEM((MAX_DEGREE, WIDTH), jnp.float32),
            pltpu.VMEM((DBUF,), jnp.int32),
            pltpu.VMEM((CPAD, WIDTH), jnp.float32),
            pltpu.VMEM((CPAD, WIDTH), jnp.float32),
            pltpu.VMEM((CPAD, WIDTH), jnp.float32),
            pltpu.VMEM((CPAD, WIDTH), jnp.float32),
            pltpu.VMEM((CPAD, WIDTH), jnp.float32),
            pltpu.VMEM((CPAD, WIDTH), jnp.float32),
            pltpu.SemaphoreType.DMA,
            pltpu.SemaphoreType.DMA,
            pltpu.SemaphoreType.DMA,
            pltpu.SemaphoreType.DMA,
            pltpu.SemaphoreType.DMA,
            pltpu.SemaphoreType.DMA,
            pltpu.SemaphoreType.DMA,
            pltpu.SemaphoreType.DMA,
        ],
        compiler_params=pltpu.CompilerParams(use_tc_tiling_on_sc=False),
    )
    return f(x, d32, scale)


# speedup vs baseline: 1.0007x; 1.0007x over previous
import jax
import jax.numpy as jnp
from jax import lax
from jax.experimental import pallas as pl
from jax.experimental.pallas import tpu as pltpu
from jax.experimental.pallas import tpu_sc as plsc

N = 100000
WIDTH = 128
MAX_DEGREE = 4
L = 16
NW = 32
RPT = N // NW
CHUNK = 125
CPAD = 128
NCHUNK = RPT // CHUNK
DBUF = 3152
DIO = 3136
GROUPS = WIDTH // L
RGRP = CPAD // L
NBUF = 3


def _sc_body(x_hbm, d_hbm, scale_hbm, out_hbm,
             scv, dv, xb0, xb1, xb2, ob0, ob1, ob2,
             in_sem0, in_sem1, in_sem2, out_sem0, out_sem1, out_sem2,
             d_sem, sc_sem):
    cid = lax.axis_index("c")
    sid = lax.axis_index("s")
    wid = sid * 2 + cid
    base = wid * RPT
    ab = jnp.minimum((base // 8) * 8, N - DIO)
    off = base - ab

    xbs = [xb0, xb1, xb2]
    obs = [ob0, ob1, ob2]
    in_sems = [in_sem0, in_sem1, in_sem2]
    out_sems = [out_sem0, out_sem1, out_sem2]

    def in_copy(b, ch):
        return pltpu.make_async_copy(
            x_hbm.at[pl.ds(base + ch * CHUNK, CHUNK)],
            xbs[b].at[pl.ds(0, CHUNK)], in_sems[b])

    def out_copy(b, ch):
        return pltpu.make_async_copy(
            obs[b].at[pl.ds(0, CHUNK)],
            out_hbm.at[pl.ds(base + ch * CHUNK, CHUNK)], out_sems[b])

    for b in range(NBUF):
        in_copy(b, b).start()

    d_dma = pltpu.make_async_copy(d_hbm.at[pl.ds(ab, DIO)],
                                  dv.at[pl.ds(0, DIO)], d_sem)
    sc_dma = pltpu.make_async_copy(scale_hbm, scv, sc_sem)
    d_dma.start()
    sc_dma.start()
    sc_dma.wait()
    esc = [[jnp.exp(scv[i, pl.ds(j * L, L)]) for j in range(GROUPS)]
           for i in range(MAX_DEGREE)]
    d_dma.wait()

    def compute(xbuf, obuf, ch):
        dbase = off + ch * CHUNK

        def grp(g, carry):
            drv = dv[pl.ds(dbase + g * 8, L)]
            for k in range(8):
                dr = drv[k]
                b0 = dr == 0
                b1 = dr == 1
                b2 = dr == 2
                r = g * 8 + k
                for j in range(GROUPS):
                    m = jnp.where(b0, esc[0][j],
                                  jnp.where(b1, esc[1][j],
                                            jnp.where(b2, esc[2][j],
                                                      esc[3][j])))
                    obuf[r, pl.ds(j * L, L)] = xbuf[r, pl.ds(j * L, L)] * m
            return carry

        lax.fori_loop(0, CPAD // 8, grp, 0)

    def round_body(i, carry):
        for b in range(NBUF):
            ch = NBUF * i + b
            in_copy(b, ch).wait()

            @pl.when(i > 0)
            def _():
                out_copy(b, ch - NBUF).wait()

            compute(xbs[b], obs[b], ch)
            out_copy(b, ch).start()

            @pl.when(ch + NBUF < NCHUNK)
            def _():
                in_copy(b, ch + NBUF).start()
        return carry

    lax.fori_loop(0, NCHUNK // NBUF, round_body, 0)

    last = NCHUNK - 1
    in_copy(0, last).wait()
    out_copy(0, last - NBUF).wait()
    compute(xbs[0], obs[0], last)
    out_copy(0, last).start()
    out_copy(0, last).wait()
    for b in range(1, NBUF):
        out_copy(b, last - NBUF + b).wait()


def kernel(x, d, scale):
    d32 = d.astype(jnp.int32)
    mesh = plsc.VectorSubcoreMesh(core_axis_name="c", subcore_axis_name="s")
    f = pl.kernel(
        _sc_body,
        out_type=jax.ShapeDtypeStruct((N, WIDTH), jnp.float32),
        mesh=mesh,
        scratch_types=[
            pltpu.VMEM((MAX_DEGREE, WIDTH), jnp.float32),
            pltpu.VMEM((DBUF,), jnp.int32),
            pltpu.VMEM((CPAD, WIDTH), jnp.float32),
            pltpu.VMEM((CPAD, WIDTH), jnp.float32),
            pltpu.VMEM((CPAD, WIDTH), jnp.float32),
            pltpu.VMEM((CPAD, WIDTH), jnp.float32),
            pltpu.VMEM((CPAD, WIDTH), jnp.float32),
            pltpu.VMEM((CPAD, WIDTH), jnp.float32),
            pltpu.SemaphoreType.DMA,
            pltpu.SemaphoreType.DMA,
            pltpu.SemaphoreType.DMA,
            pltpu.SemaphoreType.DMA,
            pltpu.SemaphoreType.DMA,
            pltpu.SemaphoreType.DMA,
            pltpu.SemaphoreType.DMA,
            pltpu.SemaphoreType.DMA,
        ],
        compiler_params=pltpu.CompilerParams(use_tc_tiling_on_sc=False),
    )
    return f(x, d32, scale)


# speedup vs baseline: 1.0037x; 1.0030x over previous
import jax
import jax.numpy as jnp
from jax import lax
from jax.experimental import pallas as pl
from jax.experimental.pallas import tpu as pltpu
from jax.experimental.pallas import tpu_sc as plsc

N = 100000
WIDTH = 128
MAX_DEGREE = 4
L = 16
NW = 32
RPT = N // NW
CHUNK = 125
CPAD = 128
NCHUNK = RPT // CHUNK
DBUF = 3152
DIO = 3136
GROUPS = WIDTH // L
RROWS = 8
NBUF = 3


def _sc_body(x_hbm, d_hbm, scale_hbm, out_hbm,
             scv, dv, xb0, xb1, xb2, ob0, ob1, ob2,
             in_sem0, in_sem1, in_sem2, out_sem0, out_sem1, out_sem2,
             d_sem, sc_sem):
    cid = lax.axis_index("c")
    sid = lax.axis_index("s")
    wid = sid * 2 + cid
    base = wid * RPT
    ab = jnp.minimum((base // 8) * 8, N - DIO)
    off = base - ab

    xbs = [xb0, xb1, xb2]
    obs = [ob0, ob1, ob2]
    in_sems = [in_sem0, in_sem1, in_sem2]
    out_sems = [out_sem0, out_sem1, out_sem2]

    def in_copy(b, ch):
        return pltpu.make_async_copy(
            x_hbm.at[pl.ds(base + ch * CHUNK, CHUNK)],
            xbs[b].at[pl.ds(0, CHUNK)], in_sems[b])

    def out_copy(b, ch):
        return pltpu.make_async_copy(
            obs[b].at[pl.ds(0, CHUNK)],
            out_hbm.at[pl.ds(base + ch * CHUNK, CHUNK)], out_sems[b])

    for b in range(NBUF):
        in_copy(b, b).start()

    d_dma = pltpu.make_async_copy(d_hbm.at[pl.ds(ab, DIO)],
                                  dv.at[pl.ds(0, DIO)], d_sem)
    sc_dma = pltpu.make_async_copy(scale_hbm, scv, sc_sem)
    d_dma.start()
    sc_dma.start()
    sc_dma.wait()
    esc = [[jnp.exp(scv[i, pl.ds(j * L, L)]) for j in range(GROUPS)]
           for i in range(MAX_DEGREE)]
    d_dma.wait()

    def compute(xbuf, obuf, ch):
        dbase = off + ch * CHUNK

        def grp(g, carry):
            drv = dv[pl.ds(dbase + g * RROWS, L)]
            for k in range(RROWS):
                dr = drv[k]
                b0 = dr == 0
                b1 = dr == 1
                b2 = dr == 2
                r = g * RROWS + k
                for j in range(GROUPS):
                    m = jnp.where(b0, esc[0][j],
                                  jnp.where(b1, esc[1][j],
                                            jnp.where(b2, esc[2][j],
                                                      esc[3][j])))
                    obuf[r, pl.ds(j * L, L)] = xbuf[r, pl.ds(j * L, L)] * m
            return carry

        lax.fori_loop(0, CPAD // RROWS, grp, 0)

    def round_body(i, carry):
        for b in range(NBUF):
            ch = NBUF * i + b
            in_copy(b, ch).wait()

            @pl.when(i > 0)
            def _():
                out_copy(b, ch - NBUF).wait()

            compute(xbs[b], obs[b], ch)
            out_copy(b, ch).start()

            @pl.when(ch + NBUF < NCHUNK)
            def _():
                in_copy(b, ch + NBUF).start()
        return carry

    lax.fori_loop(0, NCHUNK // NBUF, round_body, 0)

    last = NCHUNK - 1
    in_copy(0, last).wait()
    out_copy(0, last - NBUF).wait()
    compute(xbs[0], obs[0], last)
    out_copy(0, last).start()
    out_copy(0, last).wait()
    for b in range(1, NBUF):
        out_copy(b, last - NBUF + b).wait()


def kernel(x, d, scale):
    d32 = d.astype(jnp.int32)
    mesh = plsc.VectorSubcoreMesh(core_axis_name="c", subcore_axis_name="s")
    f = pl.kernel(
        _sc_body,
        out_type=jax.ShapeDtypeStruct((N, WIDTH), jnp.float32),
        mesh=mesh,
        scratch_types=[
            pltpu.VMEM((MAX_DEGREE, WIDTH), jnp.float32),
            pltpu.VMEM((DBUF,), jnp.int32),
            pltpu.VMEM((CPAD, WIDTH), jnp.float32),
            pltpu.VMEM((CPAD, WIDTH), jnp.float32),
            pltpu.VMEM((CPAD, WIDTH), jnp.float32),
            pltpu.VMEM((CPAD, WIDTH), jnp.float32),
            pltpu.VMEM((CPAD, WIDTH), jnp.float32),
            pltpu.VMEM((CPAD, WIDTH), jnp.float32),
            pltpu.SemaphoreType.DMA,
            pltpu.SemaphoreType.DMA,
            pltpu.SemaphoreType.DMA,
            pltpu.SemaphoreType.DMA,
            pltpu.SemaphoreType.DMA,
            pltpu.SemaphoreType.DMA,
            pltpu.SemaphoreType.DMA,
            pltpu.SemaphoreType.DMA,
        ],
        compiler_params=pltpu.CompilerParams(use_tc_tiling_on_sc=False),
    )
    return f(x, d32, scale)
